# baseline (speedup 1.0000x reference)
import functools

import jax
import jax.numpy as jnp
from jax.experimental import pallas as pl
from jax.experimental.pallas import tpu as pltpu

B, S, H = 1, 2048, 4
QK_D, V_D = 128, 128
KERNEL, STRIDE, SELECT, TOP_N, WINDOW = 32, 16, 64, 8, 256

QT = 256
KT = 256
NQT = S // QT
NKT = S // KT
NCMP = (S - KERNEL) // STRIDE + 1
NCMP_PAD = 128
NSEL = S // SELECT
SCALE = QK_D ** -0.5
NEG = -1e30


def _nsa_body(q_ref, k_ref, v_ref, wk_ref, wv_ref, wg_ref, bg_ref,
              o_ref, s_scr, ck_scr, cv_scr):
    i = pl.program_id(1)
    qs = i * QT
    q = q_ref[...]
    t = qs + jax.lax.broadcasted_iota(jnp.int32, (QT, 1), 0)

    @pl.when(i == 0)
    def _pool():
        kbf = k_ref[...].astype(jnp.bfloat16)
        vbf = v_ref[...].astype(jnp.bfloat16)
        ck_scr[...] = jax.lax.dot_general(
            wk_ref[...], kbf, (((1,), (0,)), ((), ())),
            preferred_element_type=jnp.float32)
        cv_scr[...] = jax.lax.dot_general(
            wv_ref[...], vbf, (((1,), (0,)), ((), ())),
            preferred_element_type=jnp.float32)

    cmp_k = ck_scr[...]
    cmp_v = cv_scr[...]

    s_cmp = jax.lax.dot_general(q, cmp_k, (((1,), (1,)), ((), ())),
                                preferred_element_type=jnp.float32) * SCALE
    n_iota = jax.lax.broadcasted_iota(jnp.int32, (QT, NCMP_PAD), 1)
    cmp_valid = (n_iota < NCMP) & (n_iota * STRIDE <= t)
    s_cmp = jnp.where(cmp_valid, s_cmp, NEG)
    m_c = jnp.max(s_cmp, axis=1, keepdims=True)
    e_c = jnp.where(cmp_valid, jnp.exp(s_cmp - m_c), 0.0)
    p_cmp = e_c / jnp.maximum(e_c.sum(axis=1, keepdims=True), 1e-30)
    cmp_o = jnp.dot(p_cmp, cmp_v, preferred_element_type=jnp.float32)

    sel_row = jax.lax.broadcasted_iota(jnp.int32, (NCMP_PAD, NSEL), 0)
    sel_col = jax.lax.broadcasted_iota(jnp.int32, (NCMP_PAD, NSEL), 1)
    sel_map = ((sel_row // 4 == sel_col) & (sel_row < NCMP)).astype(jnp.float32)
    p_sel = jnp.dot(p_cmp, sel_map, preferred_element_type=jnp.float32)
    m_iota = jax.lax.broadcasted_iota(jnp.int32, (QT, NSEL), 1)
    own = (m_iota == t // SELECT).astype(jnp.float32)
    first = (m_iota == 0).astype(jnp.float32)
    p_sel = p_sel + 1e6 * own + 5e5 * first
    p_sel = jnp.where(m_iota * SELECT > t, NEG, p_sel)
    xa = p_sel[:, :, None]
    xb = p_sel[:, None, :]
    ia = jax.lax.broadcasted_iota(jnp.int32, (QT, NSEL, NSEL), 1)
    ib = jax.lax.broadcasted_iota(jnp.int32, (QT, NSEL, NSEL), 2)
    beats = (xb > xa) | ((xb == xa) & (ib < ia))
    rank = beats.astype(jnp.float32).sum(axis=2)
    sel_mask = (rank < TOP_N).astype(jnp.float32)

    m_row32 = jax.lax.broadcasted_iota(jnp.int32, (NSEL, KT), 0)
    c_blk = jax.lax.broadcasted_iota(jnp.int32, (NSEL, KT), 1) // SELECT
    tc_t = jax.lax.broadcasted_iota(jnp.int32, (QT, KT), 1)

    def sel_ok(j, s_or_none):
        ej = (m_row32 == j * 4 + c_blk).astype(jnp.float32)
        mloc = jnp.dot(sel_mask, ej, preferred_element_type=jnp.float32)
        return (mloc > 0.5) & (j * KT + tc_t <= t)

    def qk_body(j, m_run):
        kb_ = k_ref[pl.ds(j * KT, KT), :]
        s = jax.lax.dot_general(q, kb_, (((1,), (1,)), ((), ())),
                                preferred_element_type=jnp.float32) * SCALE
        s_scr[:, pl.ds(j * KT, KT)] = s
        sm = jnp.where(sel_ok(j, None), s, NEG)
        return jnp.maximum(m_run, jnp.max(sm, axis=1, keepdims=True))

    nj = (i + 1) * (QT // KT)
    m_s = jax.lax.fori_loop(0, nj, qk_body,
                            jnp.full((QT, 1), NEG, jnp.float32))

    WB = QT + KT
    wstart = jnp.maximum(i * (QT // KT) - 1, 0) * KT
    s_band = s_scr[:, pl.ds(wstart, WB)]
    tc_b = wstart + jax.lax.broadcasted_iota(jnp.int32, (QT, WB), 1)
    w_ok = (tc_b <= t) & (t - tc_b <= WINDOW)
    m_w = jnp.max(jnp.where(w_ok, s_band, NEG), axis=1, keepdims=True)
    e_w = jnp.where(w_ok, jnp.exp(s_band - m_w), 0.0)
    l_w = e_w.sum(axis=1, keepdims=True)
    v_band = v_ref[pl.ds(wstart, WB), :]
    win_o = jnp.dot(e_w, v_band, preferred_element_type=jnp.float32) \
        / jnp.maximum(l_w, 1e-30)

    def pv_body(j, carry):
        acc, l = carry
        s = s_scr[:, pl.ds(j * KT, KT)]
        e = jnp.where(sel_ok(j, None), jnp.exp(s - m_s), 0.0)
        vj = v_ref[pl.ds(j * KT, KT), :]
        acc = acc + jnp.dot(e, vj, preferred_element_type=jnp.float32)
        return acc, l + e.sum(axis=1, keepdims=True)

    acc, l_s = jax.lax.fori_loop(
        0, nj, pv_body,
        (jnp.zeros((QT, V_D), jnp.float32), jnp.zeros((QT, 1), jnp.float32)))
    sel_o = acc / jnp.maximum(l_s, 1e-30)

    glog = jnp.dot(q, wg_ref[...], preferred_element_type=jnp.float32) \
        + bg_ref[...]
    g = jax.nn.sigmoid(glog)
    out = cmp_o * g[:, 0:1] + sel_o * g[:, 1:2] + win_o * g[:, 2:3]
    o_ref[...] = out


@functools.partial(jax.jit)
def _nsa_forward(q, k, v, w_cmp_k, w_cmp_v, Wg, bg):
    nb = S // STRIDE
    r_ = jnp.arange(nb)[:, None]
    c_ = jnp.arange(nb)[None, :]
    ey0 = (c_ == r_).astype(jnp.float32)[:, :, None]
    ey1 = (c_ == r_ + 1).astype(jnp.float32)[:, :, None]
    wk = (ey0 * w_cmp_k[:STRIDE] + ey1 * w_cmp_k[STRIDE:]) \
        .reshape(nb, S).astype(jnp.bfloat16)
    wv = (ey0 * w_cmp_v[:STRIDE] + ey1 * w_cmp_v[STRIDE:]) \
        .reshape(nb, S).astype(jnp.bfloat16)
    wg = jnp.zeros((QK_D, 128), jnp.float32).at[:, :3].set(Wg)
    bgp = jnp.zeros((1, 128), jnp.float32).at[0, :3].set(bg)

    q2 = q.reshape(S, H * QK_D)
    k2 = k.reshape(S, H * QK_D)
    v2 = v.reshape(S, H * V_D)

    grid = (H, NQT)
    out = pl.pallas_call(
        _nsa_body,
        grid=grid,
        in_specs=[
            pl.BlockSpec((QT, QK_D), lambda h, i: (i, h)),
            pl.BlockSpec((S, QK_D), lambda h, i: (0, h)),
            pl.BlockSpec((S, V_D), lambda h, i: (0, h)),
            pl.BlockSpec((S // STRIDE, S), lambda h, i: (0, 0)),
            pl.BlockSpec((S // STRIDE, S), lambda h, i: (0, 0)),
            pl.BlockSpec((QK_D, 128), lambda h, i: (0, 0)),
            pl.BlockSpec((1, 128), lambda h, i: (0, 0)),
        ],
        out_specs=pl.BlockSpec((QT, V_D), lambda h, i: (i, h)),
        out_shape=jax.ShapeDtypeStruct((S, H * V_D), jnp.float32),
        scratch_shapes=[pltpu.VMEM((QT, S), jnp.float32),
                        pltpu.VMEM((S // STRIDE, QK_D), jnp.float32),
                        pltpu.VMEM((S // STRIDE, V_D), jnp.float32)],
        compiler_params=pltpu.CompilerParams(
            dimension_semantics=("parallel", "arbitrary"),
        ),
    )(q2, k2, v2, wk, wv, wg, bgp)
    return out.reshape(B, S, H, V_D)


def kernel(q, k, v, w_cmp_k, w_cmp_v, Wg, bg):
    return _nsa_forward(q, k, v, w_cmp_k, w_cmp_v, Wg, bg)


# speedup vs baseline: 1.3677x; 1.3677x over previous
import functools

import jax
import jax.numpy as jnp
from jax.experimental import pallas as pl
from jax.experimental.pallas import tpu as pltpu

B, S, H = 1, 2048, 4
QK_D, V_D = 128, 128
KERNEL, STRIDE, SELECT, TOP_N, WINDOW = 32, 16, 64, 8, 256

QT = 256
KT = 256
NQT = S // QT
NKT = S // KT
NCMP = (S - KERNEL) // STRIDE + 1
NCMP_PAD = 128
NSEL = S // SELECT
SCALE = QK_D ** -0.5
NEG = -1e30


def _nsa_body(q_ref, k_ref, v_ref, wk_ref, wv_ref, wg_ref, bg_ref,
              o_ref, s_scr, ck_scr, cv_scr):
    i = pl.program_id(1)
    qs = i * QT
    q = q_ref[...]
    t = qs + jax.lax.broadcasted_iota(jnp.int32, (QT, 1), 0)

    @pl.when(i == 0)
    def _pool():
        kbf = k_ref[...].astype(jnp.bfloat16)
        vbf = v_ref[...].astype(jnp.bfloat16)
        ck_scr[...] = jax.lax.dot_general(
            wk_ref[...], kbf, (((1,), (0,)), ((), ())),
            preferred_element_type=jnp.float32)
        cv_scr[...] = jax.lax.dot_general(
            wv_ref[...], vbf, (((1,), (0,)), ((), ())),
            preferred_element_type=jnp.float32)

    cmp_k = ck_scr[...]
    cmp_v = cv_scr[...]

    s_cmp = jax.lax.dot_general(q, cmp_k, (((1,), (1,)), ((), ())),
                                preferred_element_type=jnp.float32) * SCALE
    n_iota = jax.lax.broadcasted_iota(jnp.int32, (QT, NCMP_PAD), 1)
    cmp_valid = (n_iota < NCMP) & (n_iota * STRIDE <= t)
    s_cmp = jnp.where(cmp_valid, s_cmp, NEG)
    m_c = jnp.max(s_cmp, axis=1, keepdims=True)
    e_c = jnp.where(cmp_valid, jnp.exp(s_cmp - m_c), 0.0)
    p_cmp = e_c / jnp.maximum(e_c.sum(axis=1, keepdims=True), 1e-30)
    cmp_o = jnp.dot(p_cmp, cmp_v, preferred_element_type=jnp.float32)

    sel_row = jax.lax.broadcasted_iota(jnp.int32, (NCMP_PAD, NSEL), 0)
    sel_col = jax.lax.broadcasted_iota(jnp.int32, (NCMP_PAD, NSEL), 1)
    sel_map = ((sel_row // 4 == sel_col) & (sel_row < NCMP)).astype(jnp.float32)
    p_sel = jnp.dot(p_cmp, sel_map, preferred_element_type=jnp.float32)
    m_iota = jax.lax.broadcasted_iota(jnp.int32, (QT, NSEL), 1)
    own = (m_iota == t // SELECT).astype(jnp.float32)
    first = (m_iota == 0).astype(jnp.float32)
    p_sel = p_sel + 1e6 * own + 5e5 * first
    p_sel = jnp.where(m_iota * SELECT > t, NEG, p_sel)
    sel_mask = jnp.zeros((QT, NSEL), jnp.float32)
    x = p_sel
    for _ in range(TOP_N):
        mx = jnp.max(x, axis=1, keepdims=True)
        cand = jnp.where(x == mx, m_iota, NSEL)
        fi = jnp.min(cand, axis=1, keepdims=True)
        chosen = m_iota == fi
        sel_mask = jnp.where(chosen, 1.0, sel_mask)
        x = jnp.where(chosen, -jnp.inf, x)

    m_row32 = jax.lax.broadcasted_iota(jnp.int32, (NSEL, KT), 0)
    c_blk = jax.lax.broadcasted_iota(jnp.int32, (NSEL, KT), 1) // SELECT
    tc_t = jax.lax.broadcasted_iota(jnp.int32, (QT, KT), 1)

    def sel_ok(j, s_or_none):
        ej = (m_row32 == j * 4 + c_blk).astype(jnp.float32)
        mloc = jnp.dot(sel_mask, ej, preferred_element_type=jnp.float32)
        return (mloc > 0.5) & (j * KT + tc_t <= t)

    qbf = q.astype(jnp.bfloat16)

    def qk_body(j, m_run):
        kb_ = k_ref[pl.ds(j * KT, KT), :]
        s = jax.lax.dot_general(qbf, kb_.astype(jnp.bfloat16),
                                (((1,), (1,)), ((), ())),
                                preferred_element_type=jnp.float32) * SCALE
        s_scr[:, pl.ds(j * KT, KT)] = s
        sm = jnp.where(sel_ok(j, None), s, NEG)
        return jnp.maximum(m_run, jnp.max(sm, axis=1, keepdims=True))

    nj = (i + 1) * (QT // KT)
    m_s = jax.lax.fori_loop(0, nj, qk_body,
                            jnp.full((QT, 1), NEG, jnp.float32))

    WB = QT + KT
    wstart = jnp.maximum(i * (QT // KT) - 1, 0) * KT
    s_band = s_scr[:, pl.ds(wstart, WB)]
    tc_b = wstart + jax.lax.broadcasted_iota(jnp.int32, (QT, WB), 1)
    w_ok = (tc_b <= t) & (t - tc_b <= WINDOW)
    m_w = jnp.max(jnp.where(w_ok, s_band, NEG), axis=1, keepdims=True)
    e_w = jnp.where(w_ok, jnp.exp(s_band - m_w), 0.0)
    l_w = e_w.sum(axis=1, keepdims=True)
    v_band = v_ref[pl.ds(wstart, WB), :]
    win_o = jnp.dot(e_w.astype(jnp.bfloat16), v_band.astype(jnp.bfloat16),
                    preferred_element_type=jnp.float32) \
        / jnp.maximum(l_w, 1e-30)

    def pv_body(j, carry):
        acc, l = carry
        s = s_scr[:, pl.ds(j * KT, KT)]
        e = jnp.where(sel_ok(j, None), jnp.exp(s - m_s), 0.0)
        vj = v_ref[pl.ds(j * KT, KT), :]
        acc = acc + jnp.dot(e.astype(jnp.bfloat16), vj.astype(jnp.bfloat16),
                            preferred_element_type=jnp.float32)
        return acc, l + e.sum(axis=1, keepdims=True)

    acc, l_s = jax.lax.fori_loop(
        0, nj, pv_body,
        (jnp.zeros((QT, V_D), jnp.float32), jnp.zeros((QT, 1), jnp.float32)))
    sel_o = acc / jnp.maximum(l_s, 1e-30)

    glog = jnp.dot(q, wg_ref[...], preferred_element_type=jnp.float32) \
        + bg_ref[...]
    g = jax.nn.sigmoid(glog)
    out = cmp_o * g[:, 0:1] + sel_o * g[:, 1:2] + win_o * g[:, 2:3]
    o_ref[...] = out


@functools.partial(jax.jit)
def _nsa_forward(q, k, v, w_cmp_k, w_cmp_v, Wg, bg):
    nb = S // STRIDE
    r_ = jnp.arange(nb)[:, None]
    c_ = jnp.arange(nb)[None, :]
    ey0 = (c_ == r_).astype(jnp.float32)[:, :, None]
    ey1 = (c_ == r_ + 1).astype(jnp.float32)[:, :, None]
    wk = (ey0 * w_cmp_k[:STRIDE] + ey1 * w_cmp_k[STRIDE:]) \
        .reshape(nb, S).astype(jnp.bfloat16)
    wv = (ey0 * w_cmp_v[:STRIDE] + ey1 * w_cmp_v[STRIDE:]) \
        .reshape(nb, S).astype(jnp.bfloat16)
    wg = jnp.zeros((QK_D, 128), jnp.float32).at[:, :3].set(Wg)
    bgp = jnp.zeros((1, 128), jnp.float32).at[0, :3].set(bg)

    q2 = q.reshape(S, H * QK_D)
    k2 = k.reshape(S, H * QK_D)
    v2 = v.reshape(S, H * V_D)

    grid = (H, NQT)
    out = pl.pallas_call(
        _nsa_body,
        grid=grid,
        in_specs=[
            pl.BlockSpec((QT, QK_D), lambda h, i: (i, h)),
            pl.BlockSpec((S, QK_D), lambda h, i: (0, h)),
            pl.BlockSpec((S, V_D), lambda h, i: (0, h)),
            pl.BlockSpec((S // STRIDE, S), lambda h, i: (0, 0)),
            pl.BlockSpec((S // STRIDE, S), lambda h, i: (0, 0)),
            pl.BlockSpec((QK_D, 128), lambda h, i: (0, 0)),
            pl.BlockSpec((1, 128), lambda h, i: (0, 0)),
        ],
        out_specs=pl.BlockSpec((QT, V_D), lambda h, i: (i, h)),
        out_shape=jax.ShapeDtypeStruct((S, H * V_D), jnp.float32),
        scratch_shapes=[pltpu.VMEM((QT, S), jnp.float32),
                        pltpu.VMEM((S // STRIDE, QK_D), jnp.float32),
                        pltpu.VMEM((S // STRIDE, V_D), jnp.float32)],
        compiler_params=pltpu.CompilerParams(
            dimension_semantics=("parallel", "arbitrary"),
        ),
    )(q2, k2, v2, wk, wv, wg, bgp)
    return out.reshape(B, S, H, V_D)


def kernel(q, k, v, w_cmp_k, w_cmp_v, Wg, bg):
    return _nsa_forward(q, k, v, w_cmp_k, w_cmp_v, Wg, bg)


# speedup vs baseline: 1.5221x; 1.1129x over previous
import functools

import jax
import jax.numpy as jnp
from jax.experimental import pallas as pl
from jax.experimental.pallas import tpu as pltpu

B, S, H = 1, 2048, 4
QK_D, V_D = 128, 128
KERNEL, STRIDE, SELECT, TOP_N, WINDOW = 32, 16, 64, 8, 256

QT = 256
KT = 512
NQT = S // QT
NKT = S // KT
NCMP = (S - KERNEL) // STRIDE + 1
NCMP_PAD = 128
NSEL = S // SELECT
SCALE = QK_D ** -0.5
NEG = -1e30


def _nsa_body(q_ref, k_ref, v_ref, wk_ref, wv_ref, wg_ref, bg_ref,
              o_ref, s_scr, ck_scr, cv_scr):
    i = pl.program_id(1)
    qs = i * QT
    q = q_ref[...]
    t = qs + jax.lax.broadcasted_iota(jnp.int32, (QT, 1), 0)

    @pl.when(i == 0)
    def _pool():
        kbf = k_ref[...].astype(jnp.bfloat16)
        vbf = v_ref[...].astype(jnp.bfloat16)
        ck_scr[...] = jax.lax.dot_general(
            wk_ref[...], kbf, (((1,), (0,)), ((), ())),
            preferred_element_type=jnp.float32)
        cv_scr[...] = jax.lax.dot_general(
            wv_ref[...], vbf, (((1,), (0,)), ((), ())),
            preferred_element_type=jnp.float32)

    cmp_k = ck_scr[...]
    cmp_v = cv_scr[...]

    s_cmp = jax.lax.dot_general(q, cmp_k, (((1,), (1,)), ((), ())),
                                preferred_element_type=jnp.float32) * SCALE
    n_iota = jax.lax.broadcasted_iota(jnp.int32, (QT, NCMP_PAD), 1)
    cmp_valid = (n_iota < NCMP) & (n_iota * STRIDE <= t)
    s_cmp = jnp.where(cmp_valid, s_cmp, NEG)
    m_c = jnp.max(s_cmp, axis=1, keepdims=True)
    e_c = jnp.where(cmp_valid, jnp.exp(s_cmp - m_c), 0.0)
    p_cmp = e_c / jnp.maximum(e_c.sum(axis=1, keepdims=True), 1e-30)
    cmp_o = jnp.dot(p_cmp, cmp_v, preferred_element_type=jnp.float32)

    sel_row = jax.lax.broadcasted_iota(jnp.int32, (NCMP_PAD, NSEL), 0)
    sel_col = jax.lax.broadcasted_iota(jnp.int32, (NCMP_PAD, NSEL), 1)
    sel_map = ((sel_row // 4 == sel_col) & (sel_row < NCMP)).astype(jnp.float32)
    p_sel = jnp.dot(p_cmp, sel_map, preferred_element_type=jnp.float32)
    m_iota = jax.lax.broadcasted_iota(jnp.int32, (QT, NSEL), 1)
    own = (m_iota == t // SELECT).astype(jnp.float32)
    first = (m_iota == 0).astype(jnp.float32)
    p_sel = p_sel + 1e6 * own + 5e5 * first
    p_sel = jnp.where(m_iota * SELECT > t, NEG, p_sel)
    sel_mask = jnp.zeros((QT, NSEL), jnp.float32)
    x = p_sel
    for _ in range(TOP_N):
        mx = jnp.max(x, axis=1, keepdims=True)
        cand = jnp.where(x == mx, m_iota, NSEL)
        fi = jnp.min(cand, axis=1, keepdims=True)
        chosen = m_iota == fi
        sel_mask = jnp.where(chosen, 1.0, sel_mask)
        x = jnp.where(chosen, -jnp.inf, x)

    m_row32 = jax.lax.broadcasted_iota(jnp.int32, (NSEL, KT), 0)
    c_blk = jax.lax.broadcasted_iota(jnp.int32, (NSEL, KT), 1) // SELECT
    tc_t = jax.lax.broadcasted_iota(jnp.int32, (QT, KT), 1)

    def sel_ok(j, s_or_none):
        ej = (m_row32 == j * (KT // SELECT) + c_blk).astype(jnp.float32)
        mloc = jnp.dot(sel_mask, ej, preferred_element_type=jnp.float32)
        return (mloc > 0.5) & (j * KT + tc_t <= t)

    def qk_body(j, m_run):
        kb_ = k_ref[pl.ds(j * KT, KT), :]
        s = jax.lax.dot_general(q, kb_, (((1,), (1,)), ((), ())),
                                preferred_element_type=jnp.float32) * SCALE
        s_scr[:, pl.ds(j * KT, KT)] = s
        sm = jnp.where(sel_ok(j, None), s, NEG)
        return jnp.maximum(m_run, jnp.max(sm, axis=1, keepdims=True))

    nj = ((i + 1) * QT + KT - 1) // KT
    m_s = jax.lax.fori_loop(0, nj, qk_body,
                            jnp.full((QT, 1), NEG, jnp.float32))

    WB = QT + WINDOW
    wstart = jnp.maximum(i - 1, 0) * QT
    s_band = s_scr[:, pl.ds(wstart, WB)]
    tc_b = wstart + jax.lax.broadcasted_iota(jnp.int32, (QT, WB), 1)
    w_ok = (tc_b <= t) & (t - tc_b <= WINDOW)
    m_w = jnp.max(jnp.where(w_ok, s_band, NEG), axis=1, keepdims=True)
    e_w = jnp.where(w_ok, jnp.exp(s_band - m_w), 0.0)
    l_w = e_w.sum(axis=1, keepdims=True)
    v_band = v_ref[pl.ds(wstart, WB), :]
    win_o = jnp.dot(e_w, v_band, preferred_element_type=jnp.float32) \
        / jnp.maximum(l_w, 1e-30)

    def pv_body(j, carry):
        acc, l = carry
        s = s_scr[:, pl.ds(j * KT, KT)]
        e = jnp.where(sel_ok(j, None), jnp.exp(s - m_s), 0.0)
        vj = v_ref[pl.ds(j * KT, KT), :]
        acc = acc + jnp.dot(e, vj, preferred_element_type=jnp.float32)
        return acc, l + e.sum(axis=1, keepdims=True)

    acc, l_s = jax.lax.fori_loop(
        0, nj, pv_body,
        (jnp.zeros((QT, V_D), jnp.float32), jnp.zeros((QT, 1), jnp.float32)))
    sel_o = acc / jnp.maximum(l_s, 1e-30)

    glog = jnp.dot(q, wg_ref[...], preferred_element_type=jnp.float32) \
        + bg_ref[...]
    g = jax.nn.sigmoid(glog)
    out = cmp_o * g[:, 0:1] + sel_o * g[:, 1:2] + win_o * g[:, 2:3]
    o_ref[...] = out


@functools.partial(jax.jit)
def _nsa_forward(q, k, v, w_cmp_k, w_cmp_v, Wg, bg):
    nb = S // STRIDE
    r_ = jnp.arange(nb)[:, None]
    c_ = jnp.arange(nb)[None, :]
    ey0 = (c_ == r_).astype(jnp.float32)[:, :, None]
    ey1 = (c_ == r_ + 1).astype(jnp.float32)[:, :, None]
    wk = (ey0 * w_cmp_k[:STRIDE] + ey1 * w_cmp_k[STRIDE:]) \
        .reshape(nb, S).astype(jnp.bfloat16)
    wv = (ey0 * w_cmp_v[:STRIDE] + ey1 * w_cmp_v[STRIDE:]) \
        .reshape(nb, S).astype(jnp.bfloat16)
    wg = jnp.zeros((QK_D, 128), jnp.float32).at[:, :3].set(Wg)
    bgp = jnp.zeros((1, 128), jnp.float32).at[0, :3].set(bg)

    q2 = q.reshape(S, H * QK_D)
    k2 = k.reshape(S, H * QK_D)
    v2 = v.reshape(S, H * V_D)

    grid = (H, NQT)
    out = pl.pallas_call(
        _nsa_body,
        grid=grid,
        in_specs=[
            pl.BlockSpec((QT, QK_D), lambda h, i: (i, h)),
            pl.BlockSpec((S, QK_D), lambda h, i: (0, h)),
            pl.BlockSpec((S, V_D), lambda h, i: (0, h)),
            pl.BlockSpec((S // STRIDE, S), lambda h, i: (0, 0)),
            pl.BlockSpec((S // STRIDE, S), lambda h, i: (0, 0)),
            pl.BlockSpec((QK_D, 128), lambda h, i: (0, 0)),
            pl.BlockSpec((1, 128), lambda h, i: (0, 0)),
        ],
        out_specs=pl.BlockSpec((QT, V_D), lambda h, i: (i, h)),
        out_shape=jax.ShapeDtypeStruct((S, H * V_D), jnp.float32),
        scratch_shapes=[pltpu.VMEM((QT, S), jnp.float32),
                        pltpu.VMEM((S // STRIDE, QK_D), jnp.float32),
                        pltpu.VMEM((S // STRIDE, V_D), jnp.float32)],
        compiler_params=pltpu.CompilerParams(
            dimension_semantics=("parallel", "arbitrary"),
        ),
    )(q2, k2, v2, wk, wv, wg, bgp)
    return out.reshape(B, S, H, V_D)


def kernel(q, k, v, w_cmp_k, w_cmp_v, Wg, bg):
    return _nsa_forward(q, k, v, w_cmp_k, w_cmp_v, Wg, bg)


# speedup vs baseline: 1.6187x; 1.0635x over previous
import functools

import jax
import jax.numpy as jnp
from jax.experimental import pallas as pl
from jax.experimental.pallas import tpu as pltpu

B, S, H = 1, 2048, 4
QK_D, V_D = 128, 128
KERNEL, STRIDE, SELECT, TOP_N, WINDOW = 32, 16, 64, 8, 256

QT = 256
KT = 1024
NQT = S // QT
NKT = S // KT
NCMP = (S - KERNEL) // STRIDE + 1
NCMP_PAD = 128
NSEL = S // SELECT
SCALE = QK_D ** -0.5
NEG = -1e30


def _nsa_body(q_ref, k_ref, v_ref, wk_ref, wv_ref, wg_ref, bg_ref,
              o_ref, s_scr, ck_scr, cv_scr):
    i = pl.program_id(1)
    qs = i * QT
    q = q_ref[...]
    t = qs + jax.lax.broadcasted_iota(jnp.int32, (QT, 1), 0)

    @pl.when(i == 0)
    def _pool():
        kbf = k_ref[...].astype(jnp.bfloat16)
        vbf = v_ref[...].astype(jnp.bfloat16)
        ck_scr[...] = jax.lax.dot_general(
            wk_ref[...], kbf, (((1,), (0,)), ((), ())),
            preferred_element_type=jnp.float32)
        cv_scr[...] = jax.lax.dot_general(
            wv_ref[...], vbf, (((1,), (0,)), ((), ())),
            preferred_element_type=jnp.float32)

    cmp_k = ck_scr[...]
    cmp_v = cv_scr[...]

    s_cmp = jax.lax.dot_general(q, cmp_k, (((1,), (1,)), ((), ())),
                                preferred_element_type=jnp.float32) * SCALE
    n_iota = jax.lax.broadcasted_iota(jnp.int32, (QT, NCMP_PAD), 1)
    cmp_valid = (n_iota < NCMP) & (n_iota * STRIDE <= t)
    s_cmp = jnp.where(cmp_valid, s_cmp, NEG)
    m_c = jnp.max(s_cmp, axis=1, keepdims=True)
    e_c = jnp.where(cmp_valid, jnp.exp(s_cmp - m_c), 0.0)
    p_cmp = e_c / jnp.maximum(e_c.sum(axis=1, keepdims=True), 1e-30)
    cmp_o = jnp.dot(p_cmp, cmp_v, preferred_element_type=jnp.float32)

    sel_row = jax.lax.broadcasted_iota(jnp.int32, (NCMP_PAD, NSEL), 0)
    sel_col = jax.lax.broadcasted_iota(jnp.int32, (NCMP_PAD, NSEL), 1)
    sel_map = ((sel_row // 4 == sel_col) & (sel_row < NCMP)).astype(jnp.float32)
    p_sel = jnp.dot(p_cmp, sel_map, preferred_element_type=jnp.float32)
    m_iota = jax.lax.broadcasted_iota(jnp.int32, (QT, NSEL), 1)
    own = (m_iota == t // SELECT).astype(jnp.float32)
    first = (m_iota == 0).astype(jnp.float32)
    p_sel = p_sel + 1e6 * own + 5e5 * first
    p_sel = jnp.where(m_iota * SELECT > t, NEG, p_sel)
    sel_mask = jnp.zeros((QT, NSEL), jnp.float32)
    x = p_sel
    for _ in range(TOP_N):
        mx = jnp.max(x, axis=1, keepdims=True)
        cand = jnp.where(x == mx, m_iota, NSEL)
        fi = jnp.min(cand, axis=1, keepdims=True)
        chosen = m_iota == fi
        sel_mask = jnp.where(chosen, 1.0, sel_mask)
        x = jnp.where(chosen, -jnp.inf, x)

    m_row32 = jax.lax.broadcasted_iota(jnp.int32, (NSEL, KT), 0)
    c_blk = jax.lax.broadcasted_iota(jnp.int32, (NSEL, KT), 1) // SELECT
    tc_t = jax.lax.broadcasted_iota(jnp.int32, (QT, KT), 1)

    def sel_ok(j, s_or_none):
        ej = (m_row32 == j * (KT // SELECT) + c_blk).astype(jnp.float32)
        mloc = jnp.dot(sel_mask, ej, preferred_element_type=jnp.float32)
        return (mloc > 0.5) & (j * KT + tc_t <= t)

    def qk_body(j, m_run):
        kb_ = k_ref[pl.ds(j * KT, KT), :]
        s = jax.lax.dot_general(q, kb_, (((1,), (1,)), ((), ())),
                                preferred_element_type=jnp.float32) * SCALE
        s_scr[:, pl.ds(j * KT, KT)] = s
        sm = jnp.where(sel_ok(j, None), s, NEG)
        return jnp.maximum(m_run, jnp.max(sm, axis=1, keepdims=True))

    nj = ((i + 1) * QT + KT - 1) // KT
    m_s = jax.lax.fori_loop(0, nj, qk_body,
                            jnp.full((QT, 1), NEG, jnp.float32))

    WB = QT + WINDOW
    wstart = jnp.maximum(i - 1, 0) * QT
    s_band = s_scr[:, pl.ds(wstart, WB)]
    tc_b = wstart + jax.lax.broadcasted_iota(jnp.int32, (QT, WB), 1)
    w_ok = (tc_b <= t) & (t - tc_b <= WINDOW)
    m_w = jnp.max(jnp.where(w_ok, s_band, NEG), axis=1, keepdims=True)
    e_w = jnp.where(w_ok, jnp.exp(s_band - m_w), 0.0)
    l_w = e_w.sum(axis=1, keepdims=True)
    v_band = v_ref[pl.ds(wstart, WB), :]
    win_o = jnp.dot(e_w, v_band, preferred_element_type=jnp.float32) \
        / jnp.maximum(l_w, 1e-30)

    def pv_body(j, carry):
        acc, l = carry
        s = s_scr[:, pl.ds(j * KT, KT)]
        e = jnp.where(sel_ok(j, None), jnp.exp(s - m_s), 0.0)
        vj = v_ref[pl.ds(j * KT, KT), :]
        acc = acc + jnp.dot(e, vj, preferred_element_type=jnp.float32)
        return acc, l + e.sum(axis=1, keepdims=True)

    acc, l_s = jax.lax.fori_loop(
        0, nj, pv_body,
        (jnp.zeros((QT, V_D), jnp.float32), jnp.zeros((QT, 1), jnp.float32)))
    sel_o = acc / jnp.maximum(l_s, 1e-30)

    glog = jnp.dot(q, wg_ref[...], preferred_element_type=jnp.float32) \
        + bg_ref[...]
    g = jax.nn.sigmoid(glog)
    out = cmp_o * g[:, 0:1] + sel_o * g[:, 1:2] + win_o * g[:, 2:3]
    o_ref[...] = out


@functools.partial(jax.jit)
def _nsa_forward(q, k, v, w_cmp_k, w_cmp_v, Wg, bg):
    nb = S // STRIDE
    r_ = jnp.arange(nb)[:, None]
    c_ = jnp.arange(nb)[None, :]
    ey0 = (c_ == r_).astype(jnp.float32)[:, :, None]
    ey1 = (c_ == r_ + 1).astype(jnp.float32)[:, :, None]
    wk = (ey0 * w_cmp_k[:STRIDE] + ey1 * w_cmp_k[STRIDE:]) \
        .reshape(nb, S).astype(jnp.bfloat16)
    wv = (ey0 * w_cmp_v[:STRIDE] + ey1 * w_cmp_v[STRIDE:]) \
        .reshape(nb, S).astype(jnp.bfloat16)
    wg = jnp.zeros((QK_D, 128), jnp.float32).at[:, :3].set(Wg)
    bgp = jnp.zeros((1, 128), jnp.float32).at[0, :3].set(bg)

    q2 = q.reshape(S, H * QK_D)
    k2 = k.reshape(S, H * QK_D)
    v2 = v.reshape(S, H * V_D)

    grid = (H, NQT)
    out = pl.pallas_call(
        _nsa_body,
        grid=grid,
        in_specs=[
            pl.BlockSpec((QT, QK_D), lambda h, i: (i, h)),
            pl.BlockSpec((S, QK_D), lambda h, i: (0, h)),
            pl.BlockSpec((S, V_D), lambda h, i: (0, h)),
            pl.BlockSpec((S // STRIDE, S), lambda h, i: (0, 0)),
            pl.BlockSpec((S // STRIDE, S), lambda h, i: (0, 0)),
            pl.BlockSpec((QK_D, 128), lambda h, i: (0, 0)),
            pl.BlockSpec((1, 128), lambda h, i: (0, 0)),
        ],
        out_specs=pl.BlockSpec((QT, V_D), lambda h, i: (i, h)),
        out_shape=jax.ShapeDtypeStruct((S, H * V_D), jnp.float32),
        scratch_shapes=[pltpu.VMEM((QT, S), jnp.float32),
                        pltpu.VMEM((S // STRIDE, QK_D), jnp.float32),
                        pltpu.VMEM((S // STRIDE, V_D), jnp.float32)],
        compiler_params=pltpu.CompilerParams(
            dimension_semantics=("parallel", "arbitrary"),
        ),
    )(q2, k2, v2, wk, wv, wg, bgp)
    return out.reshape(B, S, H, V_D)


def kernel(q, k, v, w_cmp_k, w_cmp_v, Wg, bg):
    return _nsa_forward(q, k, v, w_cmp_k, w_cmp_v, Wg, bg)
